# baseline (speedup 1.0000x reference)
import functools

import jax
import jax.numpy as jnp
from jax import lax
from jax.experimental import pallas as pl
from jax.experimental.pallas import tpu as pltpu
from jax.experimental.pallas import tpu_sc as plsc

N = 5000
B = 128
NB = 40
NPAD = NB * B
THR = 0.5

BS = 128
TB = NPAD // BS
UNROLL_T = 8

SC_NC = 2
SC_NS = 16
SC_L = 16
NW = SC_NC * SC_NS
CH = NPAD // NW
G_CH = CH // SC_L
G_ALL = NPAD // SC_L

_sc_mesh = plsc.VectorSubcoreMesh(core_axis_name="c", subcore_axis_name="s")


RB = 256
RNB = NPAD // RB


def _rank_body(scol, srow, rank_ref):
    jb = pl.program_id(0)
    sj = scol[...]
    jid = jb * RB + lax.broadcasted_iota(jnp.int32, (RB, 1), 0)

    def it(c, acc):
        base = pl.multiple_of(c * 8, 8)
        tile = srow[pl.ds(base, 8), :]
        for k in range(8):
            t = c * 8 + k
            si = tile[k:k + 1, :]
            iid = t * B + lax.broadcasted_iota(jnp.int32, (1, B), 1)
            prec = (si > sj) | ((si == sj) & (iid < jid))
            acc = acc + prec.astype(jnp.float32)
        return acc

    acc = lax.fori_loop(0, NB // 8, it, jnp.zeros((RB, B), jnp.float32))
    rank_ref[...] = jnp.sum(acc, axis=1, keepdims=True).astype(jnp.int32)


def _rank(scores_p):
    out = pl.pallas_call(
        _rank_body,
        grid=(RNB,),
        in_specs=[pl.BlockSpec((RB, 1), lambda b: (b, 0)),
                  pl.BlockSpec((NB, B), lambda b: (0, 0))],
        out_specs=pl.BlockSpec((RB, 1), lambda b: (b, 0)),
        out_shape=jax.ShapeDtypeStruct((NPAD, 1), jnp.int32),
    )(scores_p.reshape(NPAD, 1), scores_p.reshape(NB, B))
    return out.reshape(NPAD)


@functools.partial(
    pl.kernel,
    out_type=tuple(jax.ShapeDtypeStruct((NPAD,), jnp.float32)
                   for _ in range(4)),
    mesh=_sc_mesh,
    compiler_params=pltpu.CompilerParams(needs_layout_passes=False),
    scratch_types=[pltpu.VMEM((NPAD,), jnp.int32),
                   pltpu.VMEM((4 * N,), jnp.float32),
                   pltpu.VMEM((CH,), jnp.int32)]
    + [pltpu.VMEM((CH,), jnp.float32) for _ in range(4)],
)
def _permute_sc(rank_hbm, boxes_hbm,
                xs_hbm, ys_hbm, rs_hbm, bs_hbm,
                rank_v, bx_v,
                ord_v, xs_v, ys_v, rs_v, bs_v):
    wid = lax.axis_index("s") * SC_NC + lax.axis_index("c")
    lo = wid * CH
    pltpu.sync_copy(rank_hbm, rank_v)
    pltpu.sync_copy(boxes_hbm, bx_v)

    def scat(g, carry):
        idx = rank_v[pl.ds(g * SC_L, SC_L)]
        src = g * SC_L + lax.iota(jnp.int32, SC_L)
        m = (idx >= lo) & (idx < lo + CH)
        plsc.store_scatter(ord_v, [idx - lo], src, mask=m)
        return carry

    lax.fori_loop(0, G_ALL, scat, 0)

    def gat(g, carry):
        sl = pl.ds(g * SC_L, SC_L)
        o = ord_v[sl]
        m = o < N
        ob = o * 4
        far = jnp.full((SC_L,), 1.0e7, jnp.float32)
        far1 = far + 1.0
        xs_v[sl] = jnp.where(m, plsc.load_gather(bx_v, [ob], mask=m), far)
        ys_v[sl] = jnp.where(m, plsc.load_gather(bx_v, [ob + 1], mask=m), far)
        rs_v[sl] = jnp.where(m, plsc.load_gather(bx_v, [ob + 2], mask=m), far1)
        bs_v[sl] = jnp.where(m, plsc.load_gather(bx_v, [ob + 3], mask=m), far1)
        return carry

    lax.fori_loop(0, G_CH, gat, 0)
    pltpu.sync_copy(xs_v, xs_hbm.at[pl.ds(lo, CH)])
    pltpu.sync_copy(ys_v, ys_hbm.at[pl.ds(lo, CH)])
    pltpu.sync_copy(rs_v, rs_hbm.at[pl.ds(lo, CH)])
    pltpu.sync_copy(bs_v, bs_hbm.at[pl.ds(lo, CH)])


def _nms_body(xr, yr, rr, br, xc, yc, rc, bc, xw, yw, rw, bw, keep_ref):
    b = pl.program_id(0)

    @pl.when(b == 0)
    def _init():
        keep_ref[...] = jnp.ones((TB, BS), jnp.float32)

    xi = xc[...]
    yi = yc[...]
    ri = rc[...]
    bi = bc[...]
    ai = (ri - xi + 1.0) * (bi - yi + 1.0)

    def supp_mat(xj, yj, rj, bj):
        aj = (rj - xj + 1.0) * (bj - yj + 1.0)
        cw = jnp.minimum(ri, rj) - jnp.maximum(xi, xj) + 1.0
        ch = jnp.minimum(bi, bj) - jnp.maximum(yi, yj) + 1.0
        cross = jnp.maximum(cw, 0.0) * jnp.maximum(ch, 0.0)
        union = ai + aj - cross
        return (cross > THR * (union + 1e-6)).astype(jnp.float32)

    xj = xr[pl.ds(b, 1), :]
    yj = yr[pl.ds(b, 1), :]
    rj = rr[pl.ds(b, 1), :]
    bj = br[pl.ds(b, 1), :]
    s_bb = supp_mat(xj, yj, rj, bj)
    ii = lax.broadcasted_iota(jnp.int32, (BS, BS), 0)
    jj = lax.broadcasted_iota(jnp.int32, (BS, BS), 1)
    s_bb = s_bb * (ii < jj).astype(jnp.float32)

    init = keep_ref[pl.ds(b, 1), :]

    def cond(c):
        return c[1]

    def body(c):
        keep, _ = c
        cnt = lax.dot_general(keep, s_bb, (((1,), (0,)), ((), ())),
                              preferred_element_type=jnp.float32)
        knew = init * (cnt < 0.5).astype(jnp.float32)
        return knew, jnp.any(knew != keep)

    keep_b, _ = lax.while_loop(cond, body, (init, True))
    keep_ref[pl.ds(b, 1), :] = keep_b

    def tailc(c, carry):
        xtw = xw[pl.ds(c, 1), :]
        ytw = yw[pl.ds(c, 1), :]
        rtw = rw[pl.ds(c, 1), :]
        btw = bw[pl.ds(c, 1), :]
        supp = supp_mat(xtw, ytw, rtw, btw)
        cnt = lax.dot_general(keep_b, supp, (((1,), (0,)), ((), ())),
                              preferred_element_type=jnp.float32)
        cnt8 = cnt.reshape(8, BS)
        base = pl.multiple_of(c * 8, 8)
        old8 = keep_ref[pl.ds(base, 8), :]
        ids = (c * 8 + lax.broadcasted_iota(jnp.int32, (8, BS), 0)) * BS \
            + lax.broadcasted_iota(jnp.int32, (8, BS), 1)
        new8 = old8 * (cnt8 < 0.5).astype(jnp.float32)
        keep_ref[pl.ds(base, 8), :] = jnp.where(
            ids >= (b + 1) * BS, new8, old8)
        return carry

    lax.fori_loop((b + 1) // 8, TB // 8, tailc, 0)


def _nms_sorted(xs, ys, rs, bs):
    W = 8 * BS
    NCH = NPAD // W
    full = pl.BlockSpec((TB, BS), lambda b: (0, 0))
    col = pl.BlockSpec((BS, 1), lambda b: (b, 0))
    wide = pl.BlockSpec((NCH, W), lambda b: (0, 0))
    keep = pl.pallas_call(
        _nms_body,
        grid=(TB,),
        in_specs=[full, full, full, full, col, col, col, col,
                  wide, wide, wide, wide],
        out_specs=pl.BlockSpec((TB, BS), lambda b: (0, 0)),
        out_shape=jax.ShapeDtypeStruct((TB, BS), jnp.float32),
    )(xs.reshape(TB, BS), ys.reshape(TB, BS), rs.reshape(TB, BS),
      bs.reshape(TB, BS), xs.reshape(NPAD, 1), ys.reshape(NPAD, 1),
      rs.reshape(NPAD, 1), bs.reshape(NPAD, 1),
      xs.reshape(NCH, W), ys.reshape(NCH, W), rs.reshape(NCH, W),
      bs.reshape(NCH, W))
    return keep.reshape(NPAD)


@functools.partial(
    pl.kernel,
    out_type=jax.ShapeDtypeStruct((NPAD,), jnp.float32),
    mesh=_sc_mesh,
    compiler_params=pltpu.CompilerParams(needs_layout_passes=False),
    scratch_types=[pltpu.VMEM((NPAD,), jnp.float32),
                   pltpu.VMEM((CH,), jnp.int32),
                   pltpu.VMEM((CH,), jnp.float32),
                   pltpu.VMEM((CH,), jnp.float32)],
)
def _unpermute_sc(rank_hbm, keep_hbm, s_hbm, out_hbm, ks_v, rk_v, s_v, o_v):
    wid = lax.axis_index("s") * SC_NC + lax.axis_index("c")
    lo = wid * CH
    pltpu.sync_copy(keep_hbm, ks_v)
    pltpu.sync_copy(rank_hbm.at[pl.ds(lo, CH)], rk_v)
    pltpu.sync_copy(s_hbm.at[pl.ds(lo, CH)], s_v)

    def gat(g, carry):
        sl = pl.ds(g * SC_L, SC_L)
        idx = rk_v[sl]
        kv = plsc.load_gather(ks_v, [idx])
        o_v[sl] = s_v[sl] * kv
        return carry

    lax.fori_loop(0, G_CH, gat, 0)
    pltpu.sync_copy(o_v, out_hbm.at[pl.ds(lo, CH)])


def kernel(boxes, scores):
    pad = NPAD - N
    scores_p = jnp.concatenate(
        [scores, jnp.full((pad,), -1.0, jnp.float32)])

    rank = _rank(scores_p)
    xs, ys, rs, bs = _permute_sc(rank, boxes.reshape(-1))
    keep_sorted = _nms_sorted(xs, ys, rs, bs)
    out = _unpermute_sc(rank, keep_sorted, scores_p)
    return out[:N]


# speedup vs baseline: 1.0472x; 1.0472x over previous
import functools

import jax
import jax.numpy as jnp
from jax import lax
from jax.experimental import pallas as pl
from jax.experimental.pallas import tpu as pltpu
from jax.experimental.pallas import tpu_sc as plsc

N = 5000
B = 128
NB = 40
NPAD = NB * B
THR = 0.5

BS = 128
TB = NPAD // BS
UNROLL_T = 8

SC_NC = 2
SC_NS = 16
SC_L = 16
NW = SC_NC * SC_NS
CH = NPAD // NW
G_CH = CH // SC_L
G_ALL = NPAD // SC_L

_sc_mesh = plsc.VectorSubcoreMesh(core_axis_name="c", subcore_axis_name="s")


RB = 256
RNB = NPAD // RB


def _rank_body(scol, srow, rank_ref):
    jb = pl.program_id(0)
    sj = scol[...]
    jid = jb * RB + lax.broadcasted_iota(jnp.int32, (RB, 1), 0)

    def it(c, acc):
        base = pl.multiple_of(c * 8, 8)
        tile = srow[pl.ds(base, 8), :]
        for k in range(8):
            t = c * 8 + k
            si = tile[k:k + 1, :]
            iid = t * B + lax.broadcasted_iota(jnp.int32, (1, B), 1)
            prec = (si > sj) | ((si == sj) & (iid < jid))
            acc = acc + prec.astype(jnp.float32)
        return acc

    acc = lax.fori_loop(0, NB // 8, it, jnp.zeros((RB, B), jnp.float32))
    rank_ref[...] = jnp.sum(acc, axis=1, keepdims=True).astype(jnp.int32)


def _rank(scores_p):
    out = pl.pallas_call(
        _rank_body,
        grid=(RNB,),
        in_specs=[pl.BlockSpec((RB, 1), lambda b: (b, 0)),
                  pl.BlockSpec((NB, B), lambda b: (0, 0))],
        out_specs=pl.BlockSpec((RB, 1), lambda b: (b, 0)),
        out_shape=jax.ShapeDtypeStruct((NPAD, 1), jnp.int32),
    )(scores_p.reshape(NPAD, 1), scores_p.reshape(NB, B))
    return out.reshape(NPAD)


@functools.partial(
    pl.kernel,
    out_type=tuple(jax.ShapeDtypeStruct((NPAD,), jnp.float32)
                   for _ in range(4)),
    mesh=_sc_mesh,
    compiler_params=pltpu.CompilerParams(needs_layout_passes=False),
    scratch_types=[pltpu.VMEM((NPAD,), jnp.int32),
                   pltpu.VMEM((4 * N,), jnp.float32),
                   pltpu.VMEM((CH,), jnp.int32)]
    + [pltpu.VMEM((CH,), jnp.float32) for _ in range(4)],
)
def _permute_sc(rank_hbm, boxes_hbm,
                xs_hbm, ys_hbm, rs_hbm, bs_hbm,
                rank_v, bx_v,
                ord_v, xs_v, ys_v, rs_v, bs_v):
    wid = lax.axis_index("s") * SC_NC + lax.axis_index("c")
    lo = wid * CH
    pltpu.sync_copy(rank_hbm, rank_v)
    pltpu.sync_copy(boxes_hbm, bx_v)

    def scat(g, carry):
        idx = rank_v[pl.ds(g * SC_L, SC_L)]
        src = g * SC_L + lax.iota(jnp.int32, SC_L)
        m = (idx >= lo) & (idx < lo + CH)
        plsc.store_scatter(ord_v, [idx - lo], src, mask=m)
        return carry

    lax.fori_loop(0, G_ALL, scat, 0)

    def gat(g, carry):
        sl = pl.ds(g * SC_L, SC_L)
        o = ord_v[sl]
        m = o < N
        ob = o * 4
        far = jnp.full((SC_L,), 1.0e7, jnp.float32)
        far1 = far + 1.0
        xs_v[sl] = jnp.where(m, plsc.load_gather(bx_v, [ob], mask=m), far)
        ys_v[sl] = jnp.where(m, plsc.load_gather(bx_v, [ob + 1], mask=m), far)
        rs_v[sl] = jnp.where(m, plsc.load_gather(bx_v, [ob + 2], mask=m), far1)
        bs_v[sl] = jnp.where(m, plsc.load_gather(bx_v, [ob + 3], mask=m), far1)
        return carry

    lax.fori_loop(0, G_CH, gat, 0)
    pltpu.sync_copy(xs_v, xs_hbm.at[pl.ds(lo, CH)])
    pltpu.sync_copy(ys_v, ys_hbm.at[pl.ds(lo, CH)])
    pltpu.sync_copy(rs_v, rs_hbm.at[pl.ds(lo, CH)])
    pltpu.sync_copy(bs_v, bs_hbm.at[pl.ds(lo, CH)])


def _nms_body(xr, yr, rr, br, xc, yc, rc, bc, keep_ref):
    b = pl.program_id(0)

    @pl.when(b == 0)
    def _init():
        keep_ref[...] = jnp.ones((TB, BS), jnp.float32)

    xi = xc[...]
    yi = yc[...]
    ri1 = rc[...] + 1.0
    bi1 = bc[...] + 1.0
    ai = (ri1 - xi) * (bi1 - yi)

    def supp_mat(xj, yj, rj, bj):
        rj1 = rj + 1.0
        bj1 = bj + 1.0
        aj = (rj1 - xj) * (bj1 - yj)
        cw = jnp.minimum(ri1, rj1) - jnp.maximum(xi, xj)
        ch = jnp.minimum(bi1, bj1) - jnp.maximum(yi, yj)
        cross = jnp.maximum(cw, 0.0) * jnp.maximum(ch, 0.0)
        union = ai + aj - cross
        return (cross > THR * (union + 1e-6)).astype(jnp.float32)

    xj = xr[pl.ds(b, 1), :]
    yj = yr[pl.ds(b, 1), :]
    rj = rr[pl.ds(b, 1), :]
    bj = br[pl.ds(b, 1), :]
    s_bb = supp_mat(xj, yj, rj, bj)
    ii = lax.broadcasted_iota(jnp.int32, (BS, BS), 0)
    jj = lax.broadcasted_iota(jnp.int32, (BS, BS), 1)
    s_bb = s_bb * (ii < jj).astype(jnp.float32)

    init = keep_ref[pl.ds(b, 1), :]

    def cond(c):
        return c[1]

    def body(c):
        keep, _ = c
        cnt = lax.dot_general(keep, s_bb, (((1,), (0,)), ((), ())),
                              preferred_element_type=jnp.float32)
        k1 = init * (cnt < 0.5).astype(jnp.float32)
        cnt2 = lax.dot_general(k1, s_bb, (((1,), (0,)), ((), ())),
                               preferred_element_type=jnp.float32)
        k2 = init * (cnt2 < 0.5).astype(jnp.float32)
        return k2, jnp.any(k2 != keep)

    keep_b, _ = lax.while_loop(cond, body, (init, True))
    keep_ref[pl.ds(b, 1), :] = keep_b

    ident = (ii == jj).astype(jnp.float32)
    keep_col = lax.dot_general(ident, keep_b, (((1,), (1,)), ((), ())),
                               preferred_element_type=jnp.float32)

    def tailc(c, carry):
        base = pl.multiple_of(c * 8, 8)
        xt8 = xr[pl.ds(base, 8), :]
        yt8 = yr[pl.ds(base, 8), :]
        rt8 = rr[pl.ds(base, 8), :]
        bt8 = br[pl.ds(base, 8), :]
        old8 = keep_ref[pl.ds(base, 8), :]
        rows = []
        for k in range(8):
            t = c * 8 + k
            s_bt = supp_mat(xt8[k:k + 1, :], yt8[k:k + 1, :],
                            rt8[k:k + 1, :], bt8[k:k + 1, :])
            cnt = jnp.max(s_bt * keep_col, axis=0, keepdims=True)
            old = old8[k:k + 1, :]
            new = old * (cnt < 0.5).astype(jnp.float32)
            rows.append(jnp.where(t > b, new, old))
        keep_ref[pl.ds(base, 8), :] = jnp.concatenate(rows, axis=0)
        return carry

    lax.fori_loop((b + 1) // 8, TB // 8, tailc, 0)


def _nms_sorted(xs, ys, rs, bs):
    full = pl.BlockSpec((TB, BS), lambda b: (0, 0))
    col = pl.BlockSpec((BS, 1), lambda b: (b, 0))
    keep = pl.pallas_call(
        _nms_body,
        grid=(TB,),
        in_specs=[full, full, full, full, col, col, col, col],
        out_specs=pl.BlockSpec((TB, BS), lambda b: (0, 0)),
        out_shape=jax.ShapeDtypeStruct((TB, BS), jnp.float32),
    )(xs.reshape(TB, BS), ys.reshape(TB, BS), rs.reshape(TB, BS),
      bs.reshape(TB, BS), xs.reshape(NPAD, 1), ys.reshape(NPAD, 1),
      rs.reshape(NPAD, 1), bs.reshape(NPAD, 1))
    return keep.reshape(NPAD)


@functools.partial(
    pl.kernel,
    out_type=jax.ShapeDtypeStruct((NPAD,), jnp.float32),
    mesh=_sc_mesh,
    compiler_params=pltpu.CompilerParams(needs_layout_passes=False),
    scratch_types=[pltpu.VMEM((NPAD,), jnp.float32),
                   pltpu.VMEM((CH,), jnp.int32),
                   pltpu.VMEM((CH,), jnp.float32),
                   pltpu.VMEM((CH,), jnp.float32)],
)
def _unpermute_sc(rank_hbm, keep_hbm, s_hbm, out_hbm, ks_v, rk_v, s_v, o_v):
    wid = lax.axis_index("s") * SC_NC + lax.axis_index("c")
    lo = wid * CH
    pltpu.sync_copy(keep_hbm, ks_v)
    pltpu.sync_copy(rank_hbm.at[pl.ds(lo, CH)], rk_v)
    pltpu.sync_copy(s_hbm.at[pl.ds(lo, CH)], s_v)

    def gat(g, carry):
        sl = pl.ds(g * SC_L, SC_L)
        idx = rk_v[sl]
        kv = plsc.load_gather(ks_v, [idx])
        o_v[sl] = s_v[sl] * kv
        return carry

    lax.fori_loop(0, G_CH, gat, 0)
    pltpu.sync_copy(o_v, out_hbm.at[pl.ds(lo, CH)])


def kernel(boxes, scores):
    pad = NPAD - N
    scores_p = jnp.concatenate(
        [scores, jnp.full((pad,), -1.0, jnp.float32)])

    rank = _rank(scores_p)
    xs, ys, rs, bs = _permute_sc(rank, boxes.reshape(-1))
    keep_sorted = _nms_sorted(xs, ys, rs, bs)
    out = _unpermute_sc(rank, keep_sorted, scores_p)
    return out[:N]


# speedup vs baseline: 1.0773x; 1.0288x over previous
import functools

import jax
import jax.numpy as jnp
from jax import lax
from jax.experimental import pallas as pl
from jax.experimental.pallas import tpu as pltpu
from jax.experimental.pallas import tpu_sc as plsc

N = 5000
B = 128
NB = 40
NPAD = NB * B
THR = 0.5

BS = 128
TB = NPAD // BS
UNROLL_T = 8

SC_NC = 2
SC_NS = 16
SC_L = 16
NW = SC_NC * SC_NS
CH = NPAD // NW
G_CH = CH // SC_L
G_ALL = NPAD // SC_L

_sc_mesh = plsc.VectorSubcoreMesh(core_axis_name="c", subcore_axis_name="s")


RB = 256
RNB = NPAD // RB


def _rank_body(scol, srow, rank_ref):
    jb = pl.program_id(0)
    sj = scol[...]
    jid = jb * RB + lax.broadcasted_iota(jnp.int32, (RB, 1), 0)

    def it(c, acc):
        base = pl.multiple_of(c * 8, 8)
        tile = srow[pl.ds(base, 8), :]
        for k in range(8):
            t = c * 8 + k
            si = tile[k:k + 1, :]
            iid = t * B + lax.broadcasted_iota(jnp.int32, (1, B), 1)
            prec = (si > sj) | ((si == sj) & (iid < jid))
            acc = acc + prec.astype(jnp.float32)
        return acc

    acc = lax.fori_loop(0, NB // 8, it, jnp.zeros((RB, B), jnp.float32))
    rank_ref[...] = jnp.sum(acc, axis=1, keepdims=True).astype(jnp.int32)


def _rank(scores_p):
    out = pl.pallas_call(
        _rank_body,
        grid=(RNB,),
        in_specs=[pl.BlockSpec((RB, 1), lambda b: (b, 0)),
                  pl.BlockSpec((NB, B), lambda b: (0, 0))],
        out_specs=pl.BlockSpec((RB, 1), lambda b: (b, 0)),
        out_shape=jax.ShapeDtypeStruct((NPAD, 1), jnp.int32),
    )(scores_p.reshape(NPAD, 1), scores_p.reshape(NB, B))
    return out.reshape(NPAD)


@functools.partial(
    pl.kernel,
    out_type=tuple(jax.ShapeDtypeStruct((NPAD,), jnp.float32)
                   for _ in range(4)),
    mesh=_sc_mesh,
    compiler_params=pltpu.CompilerParams(needs_layout_passes=False),
    scratch_types=[pltpu.VMEM((NPAD,), jnp.int32),
                   pltpu.VMEM((4 * N,), jnp.float32),
                   pltpu.VMEM((CH,), jnp.int32)]
    + [pltpu.VMEM((CH,), jnp.float32) for _ in range(4)],
)
def _permute_sc(rank_hbm, boxes_hbm,
                xs_hbm, ys_hbm, rs_hbm, bs_hbm,
                rank_v, bx_v,
                ord_v, xs_v, ys_v, rs_v, bs_v):
    wid = lax.axis_index("s") * SC_NC + lax.axis_index("c")
    lo = wid * CH
    pltpu.sync_copy(rank_hbm, rank_v)
    pltpu.sync_copy(boxes_hbm, bx_v)

    def scat(g, carry):
        idx = rank_v[pl.ds(g * SC_L, SC_L)]
        src = g * SC_L + lax.iota(jnp.int32, SC_L)
        m = (idx >= lo) & (idx < lo + CH)
        plsc.store_scatter(ord_v, [idx - lo], src, mask=m)
        return carry

    lax.fori_loop(0, G_ALL, scat, 0)

    def gat(g, carry):
        sl = pl.ds(g * SC_L, SC_L)
        o = ord_v[sl]
        m = o < N
        ob = o * 4
        far = jnp.full((SC_L,), 1.0e7, jnp.float32)
        far1 = far + 1.0
        xs_v[sl] = jnp.where(m, plsc.load_gather(bx_v, [ob], mask=m), far)
        ys_v[sl] = jnp.where(m, plsc.load_gather(bx_v, [ob + 1], mask=m), far)
        rs_v[sl] = jnp.where(m, plsc.load_gather(bx_v, [ob + 2], mask=m), far1)
        bs_v[sl] = jnp.where(m, plsc.load_gather(bx_v, [ob + 3], mask=m), far1)
        return carry

    lax.fori_loop(0, G_CH, gat, 0)
    pltpu.sync_copy(xs_v, xs_hbm.at[pl.ds(lo, CH)])
    pltpu.sync_copy(ys_v, ys_hbm.at[pl.ds(lo, CH)])
    pltpu.sync_copy(rs_v, rs_hbm.at[pl.ds(lo, CH)])
    pltpu.sync_copy(bs_v, bs_hbm.at[pl.ds(lo, CH)])


def _nms_body(xr, yr, rr, br, xc, yc, rc, bc, keep_ref):
    b = pl.program_id(0)

    @pl.when(b == 0)
    def _init():
        keep_ref[...] = jnp.ones((TB, BS), jnp.float32)

    xi = xc[...]
    yi = yc[...]
    ri1 = rc[...] + 1.0
    bi1 = bc[...] + 1.0
    ai = (ri1 - xi) * (bi1 - yi)

    def supp_mat(xj, yj, rj, bj):
        rj1 = rj + 1.0
        bj1 = bj + 1.0
        aj = (rj1 - xj) * (bj1 - yj)
        cw = jnp.minimum(ri1, rj1) - jnp.maximum(xi, xj)
        ch = jnp.minimum(bi1, bj1) - jnp.maximum(yi, yj)
        cross = jnp.maximum(cw, 0.0) * jnp.maximum(ch, 0.0)
        union = ai + aj - cross
        return (cross > THR * (union + 1e-6)).astype(jnp.float32)

    xj = xr[pl.ds(b, 1), :]
    yj = yr[pl.ds(b, 1), :]
    rj = rr[pl.ds(b, 1), :]
    bj = br[pl.ds(b, 1), :]
    s_bb = supp_mat(xj, yj, rj, bj)
    ii = lax.broadcasted_iota(jnp.int32, (BS, BS), 0)
    jj = lax.broadcasted_iota(jnp.int32, (BS, BS), 1)
    s_bb = s_bb * (ii < jj).astype(jnp.float32)

    init = keep_ref[pl.ds(b, 1), :]

    def cond(c):
        return c[1]

    def body(c):
        keep, _ = c
        cnt = lax.dot_general(keep, s_bb, (((1,), (0,)), ((), ())),
                              preferred_element_type=jnp.float32)
        knew = init * (cnt < 0.5).astype(jnp.float32)
        return knew, jnp.any(knew != keep)

    keep_b, _ = lax.while_loop(cond, body, (init, True))
    keep_ref[pl.ds(b, 1), :] = keep_b

    ident = (ii == jj).astype(jnp.float32)
    keep_col = lax.dot_general(ident, keep_b, (((1,), (1,)), ((), ())),
                               preferred_element_type=jnp.float32)

    def tailc(c, carry):
        base = pl.multiple_of(c * 8, 8)
        xt8 = xr[pl.ds(base, 8), :]
        yt8 = yr[pl.ds(base, 8), :]
        rt8 = rr[pl.ds(base, 8), :]
        bt8 = br[pl.ds(base, 8), :]
        old8 = keep_ref[pl.ds(base, 8), :]
        rows = []
        for k in range(8):
            t = c * 8 + k
            s_bt = supp_mat(xt8[k:k + 1, :], yt8[k:k + 1, :],
                            rt8[k:k + 1, :], bt8[k:k + 1, :])
            cnt = jnp.max(s_bt * keep_col, axis=0, keepdims=True)
            old = old8[k:k + 1, :]
            new = old * (cnt < 0.5).astype(jnp.float32)
            rows.append(jnp.where(t > b, new, old))
        keep_ref[pl.ds(base, 8), :] = jnp.concatenate(rows, axis=0)
        return carry

    lax.fori_loop((b + 1) // 8, TB // 8, tailc, 0)


def _nms_sorted(xs, ys, rs, bs):
    full = pl.BlockSpec((TB, BS), lambda b: (0, 0))
    col = pl.BlockSpec((BS, 1), lambda b: (b, 0))
    keep = pl.pallas_call(
        _nms_body,
        grid=(TB,),
        in_specs=[full, full, full, full, col, col, col, col],
        out_specs=pl.BlockSpec((TB, BS), lambda b: (0, 0)),
        out_shape=jax.ShapeDtypeStruct((TB, BS), jnp.float32),
    )(xs.reshape(TB, BS), ys.reshape(TB, BS), rs.reshape(TB, BS),
      bs.reshape(TB, BS), xs.reshape(NPAD, 1), ys.reshape(NPAD, 1),
      rs.reshape(NPAD, 1), bs.reshape(NPAD, 1))
    return keep.reshape(NPAD)


@functools.partial(
    pl.kernel,
    out_type=jax.ShapeDtypeStruct((NPAD,), jnp.float32),
    mesh=_sc_mesh,
    compiler_params=pltpu.CompilerParams(needs_layout_passes=False),
    scratch_types=[pltpu.VMEM((NPAD,), jnp.float32),
                   pltpu.VMEM((CH,), jnp.int32),
                   pltpu.VMEM((CH,), jnp.float32),
                   pltpu.VMEM((CH,), jnp.float32)],
)
def _unpermute_sc(rank_hbm, keep_hbm, s_hbm, out_hbm, ks_v, rk_v, s_v, o_v):
    wid = lax.axis_index("s") * SC_NC + lax.axis_index("c")
    lo = wid * CH
    pltpu.sync_copy(keep_hbm, ks_v)
    pltpu.sync_copy(rank_hbm.at[pl.ds(lo, CH)], rk_v)
    pltpu.sync_copy(s_hbm.at[pl.ds(lo, CH)], s_v)

    def gat(g, carry):
        sl = pl.ds(g * SC_L, SC_L)
        idx = rk_v[sl]
        kv = plsc.load_gather(ks_v, [idx])
        o_v[sl] = s_v[sl] * kv
        return carry

    lax.fori_loop(0, G_CH, gat, 0)
    pltpu.sync_copy(o_v, out_hbm.at[pl.ds(lo, CH)])


def kernel(boxes, scores):
    pad = NPAD - N
    scores_p = jnp.concatenate(
        [scores, jnp.full((pad,), -1.0, jnp.float32)])

    rank = _rank(scores_p)
    xs, ys, rs, bs = _permute_sc(rank, boxes.reshape(-1))
    keep_sorted = _nms_sorted(xs, ys, rs, bs)
    out = _unpermute_sc(rank, keep_sorted, scores_p)
    return out[:N]


# speedup vs baseline: 1.0900x; 1.0117x over previous
import functools

import jax
import jax.numpy as jnp
from jax import lax
from jax.experimental import pallas as pl
from jax.experimental.pallas import tpu as pltpu
from jax.experimental.pallas import tpu_sc as plsc

N = 5000
B = 128
NB = 40
NPAD = NB * B
THR = 0.5

BS = 128
TB = NPAD // BS
UNROLL_T = 8

SC_NC = 2
SC_NS = 16
SC_L = 16
NW = SC_NC * SC_NS
CH = NPAD // NW
G_CH = CH // SC_L
G_ALL = NPAD // SC_L

_sc_mesh = plsc.VectorSubcoreMesh(core_axis_name="c", subcore_axis_name="s")


RB = 256
RNB = NPAD // RB


def _rank_body(scol, srow, rank_ref):
    jb = pl.program_id(0)
    sj = scol[...]
    jid = jb * RB + lax.broadcasted_iota(jnp.int32, (RB, 1), 0)

    def it(c, accs):
        a0, a1 = accs
        base = pl.multiple_of(c * 8, 8)
        tile = srow[pl.ds(base, 8), :]
        for k in range(8):
            t = c * 8 + k
            si = tile[k:k + 1, :]
            iid = t * B + lax.broadcasted_iota(jnp.int32, (1, B), 1)
            prec = (si > sj) | ((si == sj) & (iid < jid))
            if k % 2 == 0:
                a0 = a0 + prec.astype(jnp.float32)
            else:
                a1 = a1 + prec.astype(jnp.float32)
        return a0, a1

    z = jnp.zeros((RB, B), jnp.float32)
    a0, a1 = lax.fori_loop(0, NB // 8, it, (z, z))
    rank_ref[...] = jnp.sum(a0 + a1, axis=1, keepdims=True).astype(jnp.int32)


def _rank(scores_p):
    out = pl.pallas_call(
        _rank_body,
        grid=(RNB,),
        in_specs=[pl.BlockSpec((RB, 1), lambda b: (b, 0)),
                  pl.BlockSpec((NB, B), lambda b: (0, 0))],
        out_specs=pl.BlockSpec((RB, 1), lambda b: (b, 0)),
        out_shape=jax.ShapeDtypeStruct((NPAD, 1), jnp.int32),
    )(scores_p.reshape(NPAD, 1), scores_p.reshape(NB, B))
    return out.reshape(NPAD)


@functools.partial(
    pl.kernel,
    out_type=tuple(jax.ShapeDtypeStruct((NPAD,), jnp.float32)
                   for _ in range(4)),
    mesh=_sc_mesh,
    compiler_params=pltpu.CompilerParams(needs_layout_passes=False),
    scratch_types=[pltpu.VMEM((NPAD,), jnp.int32),
                   pltpu.VMEM((4 * N,), jnp.float32),
                   pltpu.VMEM((CH,), jnp.int32)]
    + [pltpu.VMEM((CH,), jnp.float32) for _ in range(4)],
)
def _permute_sc(rank_hbm, boxes_hbm,
                xs_hbm, ys_hbm, rs_hbm, bs_hbm,
                rank_v, bx_v,
                ord_v, xs_v, ys_v, rs_v, bs_v):
    wid = lax.axis_index("s") * SC_NC + lax.axis_index("c")
    lo = wid * CH
    pltpu.sync_copy(rank_hbm, rank_v)
    pltpu.sync_copy(boxes_hbm, bx_v)

    def scat(g, carry):
        idx = rank_v[pl.ds(g * SC_L, SC_L)]
        src = g * SC_L + lax.iota(jnp.int32, SC_L)
        m = (idx >= lo) & (idx < lo + CH)
        plsc.store_scatter(ord_v, [idx - lo], src, mask=m)
        return carry

    lax.fori_loop(0, G_ALL, scat, 0)

    def gat(g, carry):
        sl = pl.ds(g * SC_L, SC_L)
        o = ord_v[sl]
        m = o < N
        ob = o * 4
        far = jnp.full((SC_L,), 1.0e7, jnp.float32)
        far1 = far + 1.0
        xs_v[sl] = jnp.where(m, plsc.load_gather(bx_v, [ob], mask=m), far)
        ys_v[sl] = jnp.where(m, plsc.load_gather(bx_v, [ob + 1], mask=m), far)
        rs_v[sl] = jnp.where(m, plsc.load_gather(bx_v, [ob + 2], mask=m), far1)
        bs_v[sl] = jnp.where(m, plsc.load_gather(bx_v, [ob + 3], mask=m), far1)
        return carry

    lax.fori_loop(0, G_CH, gat, 0)
    pltpu.sync_copy(xs_v, xs_hbm.at[pl.ds(lo, CH)])
    pltpu.sync_copy(ys_v, ys_hbm.at[pl.ds(lo, CH)])
    pltpu.sync_copy(rs_v, rs_hbm.at[pl.ds(lo, CH)])
    pltpu.sync_copy(bs_v, bs_hbm.at[pl.ds(lo, CH)])


def _nms_body(xr, yr, rr, br, xc, yc, rc, bc, keep_ref):
    b = pl.program_id(0)

    @pl.when(b == 0)
    def _init():
        keep_ref[...] = jnp.ones((TB, BS), jnp.float32)

    xi = xc[...]
    yi = yc[...]
    ri1 = rc[...] + 1.0
    bi1 = bc[...] + 1.0
    ai = (ri1 - xi) * (bi1 - yi)

    def supp_mat(xj, yj, rj, bj):
        rj1 = rj + 1.0
        bj1 = bj + 1.0
        aj = (rj1 - xj) * (bj1 - yj)
        cw = jnp.minimum(ri1, rj1) - jnp.maximum(xi, xj)
        ch = jnp.minimum(bi1, bj1) - jnp.maximum(yi, yj)
        cross = jnp.maximum(cw, 0.0) * jnp.maximum(ch, 0.0)
        union = ai + aj - cross
        return (cross > THR * (union + 1e-6)).astype(jnp.float32)

    xj = xr[pl.ds(b, 1), :]
    yj = yr[pl.ds(b, 1), :]
    rj = rr[pl.ds(b, 1), :]
    bj = br[pl.ds(b, 1), :]
    s_bb = supp_mat(xj, yj, rj, bj)
    ii = lax.broadcasted_iota(jnp.int32, (BS, BS), 0)
    jj = lax.broadcasted_iota(jnp.int32, (BS, BS), 1)
    s_bb = s_bb * (ii < jj).astype(jnp.float32)

    init = keep_ref[pl.ds(b, 1), :]

    def cond(c):
        return c[1]

    def body(c):
        keep, _ = c
        cnt = lax.dot_general(keep, s_bb, (((1,), (0,)), ((), ())),
                              preferred_element_type=jnp.float32)
        knew = init * (cnt < 0.5).astype(jnp.float32)
        return knew, jnp.any(knew != keep)

    keep_b, _ = lax.while_loop(cond, body, (init, True))
    keep_ref[pl.ds(b, 1), :] = keep_b

    ident = (ii == jj).astype(jnp.float32)
    keep_col = lax.dot_general(ident, keep_b, (((1,), (1,)), ((), ())),
                               preferred_element_type=jnp.float32)

    def tailc(c, carry):
        base = pl.multiple_of(c * 8, 8)
        xt8 = xr[pl.ds(base, 8), :]
        yt8 = yr[pl.ds(base, 8), :]
        rt8 = rr[pl.ds(base, 8), :]
        bt8 = br[pl.ds(base, 8), :]
        old8 = keep_ref[pl.ds(base, 8), :]
        rows = []
        for k in range(8):
            t = c * 8 + k
            s_bt = supp_mat(xt8[k:k + 1, :], yt8[k:k + 1, :],
                            rt8[k:k + 1, :], bt8[k:k + 1, :])
            cnt = jnp.max(s_bt * keep_col, axis=0, keepdims=True)
            old = old8[k:k + 1, :]
            new = old * (cnt < 0.5).astype(jnp.float32)
            rows.append(jnp.where(t > b, new, old))
        keep_ref[pl.ds(base, 8), :] = jnp.concatenate(rows, axis=0)
        return carry

    lax.fori_loop((b + 1) // 8, TB // 8, tailc, 0)


def _nms_sorted(xs, ys, rs, bs):
    full = pl.BlockSpec((TB, BS), lambda b: (0, 0))
    col = pl.BlockSpec((BS, 1), lambda b: (b, 0))
    keep = pl.pallas_call(
        _nms_body,
        grid=(TB,),
        in_specs=[full, full, full, full, col, col, col, col],
        out_specs=pl.BlockSpec((TB, BS), lambda b: (0, 0)),
        out_shape=jax.ShapeDtypeStruct((TB, BS), jnp.float32),
    )(xs.reshape(TB, BS), ys.reshape(TB, BS), rs.reshape(TB, BS),
      bs.reshape(TB, BS), xs.reshape(NPAD, 1), ys.reshape(NPAD, 1),
      rs.reshape(NPAD, 1), bs.reshape(NPAD, 1))
    return keep.reshape(NPAD)


NT_FULL = N // CH
NREM = N - NT_FULL * CH


@functools.partial(
    pl.kernel,
    out_type=jax.ShapeDtypeStruct((N,), jnp.float32),
    mesh=_sc_mesh,
    compiler_params=pltpu.CompilerParams(needs_layout_passes=False),
    scratch_types=[pltpu.VMEM((NPAD,), jnp.float32),
                   pltpu.VMEM((CH,), jnp.int32),
                   pltpu.VMEM((CH,), jnp.float32),
                   pltpu.VMEM((CH,), jnp.float32)],
)
def _unpermute_sc(rank_hbm, keep_hbm, s_hbm, out_hbm, ks_v, rk_v, s_v, o_v):
    wid = lax.axis_index("s") * SC_NC + lax.axis_index("c")
    lo = wid * CH
    pltpu.sync_copy(keep_hbm, ks_v)
    pltpu.sync_copy(rank_hbm.at[pl.ds(lo, CH)], rk_v)
    pltpu.sync_copy(s_hbm.at[pl.ds(lo, CH)], s_v)

    def gat(g, carry):
        sl = pl.ds(g * SC_L, SC_L)
        idx = rk_v[sl]
        kv = plsc.load_gather(ks_v, [idx])
        o_v[sl] = s_v[sl] * kv
        return carry

    lax.fori_loop(0, G_CH, gat, 0)

    @pl.when(wid < NT_FULL)
    def _full():
        pltpu.sync_copy(o_v, out_hbm.at[pl.ds(lo, CH)])

    @pl.when(wid == NT_FULL)
    def _partial():
        pltpu.sync_copy(o_v.at[pl.ds(0, NREM)], out_hbm.at[pl.ds(lo, NREM)])


def kernel(boxes, scores):
    pad = NPAD - N
    scores_p = jnp.concatenate(
        [scores, jnp.full((pad,), -1.0, jnp.float32)])

    rank = _rank(scores_p)
    xs, ys, rs, bs = _permute_sc(rank, boxes.reshape(-1))
    keep_sorted = _nms_sorted(xs, ys, rs, bs)
    return _unpermute_sc(rank, keep_sorted, scores_p)
